# baseline (speedup 1.0000x reference)
import functools

import jax
import jax.numpy as jnp
from jax import lax
from jax.experimental import pallas as pl
from jax.experimental.pallas import tpu as pltpu
from jax.experimental.pallas import tpu_sc as plsc

TOKENS = 8192
HIDDEN = 2048
N_EXPERTS = 64
N_GROUP = 8
GROUP_SIZE = N_EXPERTS // N_GROUP
TOPK_GROUP = 3
TOP_K = 8

L = 16
N_WORKERS = 32

TOKEN_BLOCK = 1024

_SORT_CES = [
    (0, 1), (2, 3), (4, 5), (6, 7),
    (0, 2), (1, 3), (4, 6), (5, 7),
    (1, 2), (5, 6),
    (0, 4), (1, 5), (2, 6), (3, 7),
    (2, 4), (3, 5),
    (1, 2), (3, 4), (5, 6),
]


def _sorted_logits_body(hs_ref, w_ref, out_ref):
    permuted = lax.dot_general(
        hs_ref[...], w_ref[...],
        (((1,), (1,)), ((), ())),
        preferred_element_type=jnp.float32,
    )
    s = [
        permuted[:, j * N_GROUP:(j + 1) * N_GROUP]
        for j in range(GROUP_SIZE)
    ]
    for i, j in _SORT_CES:
        a, b = s[i], s[j]
        s[i] = jnp.maximum(a, b)
        s[j] = jnp.minimum(a, b)
    out_ref[...] = jnp.concatenate(s, axis=1)


def _compute_sorted_logits(hidden_states, gate_w):
    n_tok = hidden_states.shape[0]
    grid = (n_tok // TOKEN_BLOCK,)
    dst = jnp.arange(N_EXPERTS)
    w_perm = jnp.take(gate_w, (dst % N_GROUP) * GROUP_SIZE + dst // N_GROUP,
                      axis=0)
    return pl.pallas_call(
        _sorted_logits_body,
        grid=grid,
        in_specs=[
            pl.BlockSpec((TOKEN_BLOCK, HIDDEN), lambda i: (i, 0)),
            pl.BlockSpec((N_EXPERTS, HIDDEN), lambda i: (0, 0)),
        ],
        out_specs=pl.BlockSpec((TOKEN_BLOCK, N_EXPERTS), lambda i: (i, 0)),
        out_shape=jax.ShapeDtypeStruct((n_tok, N_EXPERTS), jnp.float32),
    )(hidden_states, w_perm)


def _splat_i32(v):
    return jnp.full((L,), v, jnp.int32)


def _tree_max(vals):
    vals = list(vals)
    while len(vals) > 1:
        nxt = [jnp.maximum(a, b) for a, b in zip(vals[0::2], vals[1::2])]
        if len(vals) % 2:
            nxt.append(vals[-1])
        vals = nxt
    return vals[0]


def _tree_min(vals):
    vals = list(vals)
    while len(vals) > 1:
        nxt = [jnp.minimum(a, b) for a, b in zip(vals[0::2], vals[1::2])]
        if len(vals) % 2:
            nxt.append(vals[-1])
        vals = nxt
    return vals[0]


def _make_route_body(tok_per_w, n_batch):
  def _route_body(sorted_hbm, out_hbm, lg_v, out_v):
    wid = lax.axis_index("s") * 2 + lax.axis_index("c")
    base = wid * tok_per_w
    pltpu.sync_copy(sorted_hbm.at[pl.ds(base, tok_per_w)], lg_v)

    lane = jnp.arange(L, dtype=jnp.int32)
    neg_inf = jnp.full((L,), -jnp.inf, jnp.float32)

    def one_batch(row):

        g = [plsc.load_gather(lg_v, [row, _splat_i32(k)])
             for k in range(N_GROUP)]

        sel = []
        for _ in range(TOPK_GROUP):
            m = _tree_max(g)
            idx = _tree_min([
                jnp.where(g[k] == m, _splat_i32(k), _splat_i32(N_GROUP))
                for k in range(N_GROUP)
            ])
            sel.append(idx)
            for k in range(N_GROUP):
                g[k] = jnp.where(idx == k, neg_inf, g[k])

        p = [_splat_i32(0), _splat_i32(0), _splat_i32(0)]
        top = []
        for r in range(TOP_K):
            h = [
                plsc.load_gather(lg_v, [row, p[i] * GROUP_SIZE + sel[i]])
                for i in range(TOPK_GROUP)
            ]
            m = jnp.maximum(jnp.maximum(h[0], h[1]), h[2])
            top.append(m)
            if r < TOP_K - 1:
                t0 = h[0] == m
                t1 = jnp.logical_and(h[1] == m, jnp.logical_not(t0))
                t2 = jnp.logical_and(jnp.logical_not(t0),
                                     jnp.logical_not(t1))
                one = _splat_i32(1)
                zero = _splat_i32(0)
                p[0] = p[0] + jnp.where(t0, one, zero)
                p[1] = p[1] + jnp.where(t1, one, zero)
                p[2] = p[2] + jnp.where(t2, one, zero)

        es = [jnp.exp(t - top[0]) for t in top]
        s = es[0]
        for r in range(1, TOP_K):
            s = s + es[r]
        for r in range(TOP_K):
            plsc.store_scatter(out_v, [row, _splat_i32(r)], es[r] / s)

    def batch(b, carry):
        one_batch(b * (2 * L) + lane)
        one_batch(b * (2 * L) + L + lane)
        return carry

    lax.fori_loop(0, n_batch // 2, batch, 0)
    pltpu.sync_copy(out_v, out_hbm.at[pl.ds(base, tok_per_w)])

  return _route_body


def _route(sorted_logits):
    n_tok = sorted_logits.shape[0]
    tok_per_w = n_tok // N_WORKERS
    n_batch = tok_per_w // L
    mesh = plsc.VectorSubcoreMesh(core_axis_name="c", subcore_axis_name="s")
    f = functools.partial(
        pl.kernel,
        mesh=mesh,
        out_type=jax.ShapeDtypeStruct((n_tok, TOP_K), jnp.float32),
        scratch_types=[
            pltpu.VMEM((tok_per_w, N_EXPERTS), jnp.float32),
            pltpu.VMEM((tok_per_w, TOP_K), jnp.float32),
        ],
        compiler_params=pltpu.CompilerParams(needs_layout_passes=False),
    )(_make_route_body(tok_per_w, n_batch))
    return f(sorted_logits)


def kernel(hidden_states, kernel):
    sorted_logits = _compute_sorted_logits(hidden_states, kernel)
    return _route(sorted_logits)


# speedup vs baseline: 1.0254x; 1.0254x over previous
import functools

import jax
import jax.numpy as jnp
from jax import lax
from jax.experimental import pallas as pl
from jax.experimental.pallas import tpu as pltpu
from jax.experimental.pallas import tpu_sc as plsc

TOKENS = 8192
HIDDEN = 2048
N_EXPERTS = 64
N_GROUP = 8
GROUP_SIZE = N_EXPERTS // N_GROUP
TOPK_GROUP = 3
TOP_K = 8
N_CAND = TOPK_GROUP * GROUP_SIZE

L = 16
N_WORKERS = 32

TOKEN_BLOCK = 1024


def _logits_body(hs_ref, w_ref, out_ref, gmax_ref):
    logits = lax.dot_general(
        hs_ref[...], w_ref[...],
        (((1,), (1,)), ((), ())),
        preferred_element_type=jnp.float32,
    )
    out_ref[...] = logits
    parts = [
        jnp.max(logits[:, k * GROUP_SIZE:(k + 1) * GROUP_SIZE],
                axis=1, keepdims=True)
        for k in range(N_GROUP)
    ]
    gmax_ref[...] = jnp.concatenate(parts, axis=1)


def _compute_logits(hidden_states, gate_w):
    n_tok = hidden_states.shape[0]
    grid = (n_tok // TOKEN_BLOCK,)
    return pl.pallas_call(
        _logits_body,
        grid=grid,
        in_specs=[
            pl.BlockSpec((TOKEN_BLOCK, HIDDEN), lambda i: (i, 0)),
            pl.BlockSpec((N_EXPERTS, HIDDEN), lambda i: (0, 0)),
        ],
        out_specs=[
            pl.BlockSpec((TOKEN_BLOCK, N_EXPERTS), lambda i: (i, 0)),
            pl.BlockSpec((TOKEN_BLOCK, N_GROUP), lambda i: (i, 0)),
        ],
        out_shape=[
            jax.ShapeDtypeStruct((n_tok, N_EXPERTS), jnp.float32),
            jax.ShapeDtypeStruct((n_tok, N_GROUP), jnp.float32),
        ],
    )(hidden_states, gate_w)


def _splat_i32(v):
    return jnp.full((L,), v, jnp.int32)


def _tree_max(vals):
    vals = list(vals)
    while len(vals) > 1:
        nxt = [jnp.maximum(a, b) for a, b in zip(vals[0::2], vals[1::2])]
        if len(vals) % 2:
            nxt.append(vals[-1])
        vals = nxt
    return vals[0]


def _tree_min(vals):
    vals = list(vals)
    while len(vals) > 1:
        nxt = [jnp.minimum(a, b) for a, b in zip(vals[0::2], vals[1::2])]
        if len(vals) % 2:
            nxt.append(vals[-1])
        vals = nxt
    return vals[0]


def _to_sortable(bits):
    t = lax.shift_right_arithmetic(bits, _splat_i32(31)) & _splat_i32(
        0x7FFFFFFF)
    return bits ^ t


def _make_route_body(tok_per_w, n_batch):
  def _route_body(logits_hbm, gmax_hbm, out_hbm, lg_v, gm_v, out_v):
    wid = lax.axis_index("s") * 2 + lax.axis_index("c")
    base = wid * tok_per_w
    pltpu.sync_copy(logits_hbm.at[pl.ds(base, tok_per_w)], lg_v)
    pltpu.sync_copy(gmax_hbm.at[pl.ds(base, tok_per_w)], gm_v)

    lane = jnp.arange(L, dtype=jnp.int32)
    neg_inf = jnp.full((L,), -jnp.inf, jnp.float32)
    int_min = jnp.full((L,), -0x80000000, jnp.int32)

    def batch(b, carry):
        row = b * L + lane

        g = [plsc.load_gather(gm_v, [row, _splat_i32(k)])
             for k in range(N_GROUP)]

        sel = []
        for _ in range(TOPK_GROUP):
            m = _tree_max(g)
            idx = _tree_min([
                jnp.where(g[k] == m, _splat_i32(k), _splat_i32(N_GROUP))
                for k in range(N_GROUP)
            ])
            sel.append(idx)
            for k in range(N_GROUP):
                g[k] = jnp.where(idx == k, neg_inf, g[k])

        cands = []
        for c in range(N_CAND):
            col = sel[c // GROUP_SIZE] * GROUP_SIZE + (c % GROUP_SIZE)
            cands.append(plsc.load_gather(lg_v, [row, col]))

        keys = []
        for c, v in enumerate(cands):
            s = _to_sortable(lax.bitcast_convert_type(v, jnp.int32))
            keys.append((s & _splat_i32(-32)) | _splat_i32(c))
        top = []
        for _ in range(TOP_K):
            m = _tree_max(keys)
            for c in range(N_CAND):
                keys[c] = jnp.where(keys[c] == m, int_min, keys[c])
            top.append(lax.bitcast_convert_type(
                _to_sortable(m & _splat_i32(-32)), jnp.float32))

        es = [jnp.exp(t - top[0]) for t in top]
        s = es[0]
        for r in range(1, TOP_K):
            s = s + es[r]
        for r in range(TOP_K):
            plsc.store_scatter(out_v, [row, _splat_i32(r)], es[r] / s)

        return carry

    lax.fori_loop(0, n_batch, batch, 0)
    pltpu.sync_copy(out_v, out_hbm.at[pl.ds(base, tok_per_w)])

  return _route_body


def _route(logits, gmax):
    n_tok = logits.shape[0]
    tok_per_w = n_tok // N_WORKERS
    n_batch = tok_per_w // L
    mesh = plsc.VectorSubcoreMesh(core_axis_name="c", subcore_axis_name="s")
    f = functools.partial(
        pl.kernel,
        mesh=mesh,
        out_type=jax.ShapeDtypeStruct((n_tok, TOP_K), jnp.float32),
        scratch_types=[
            pltpu.VMEM((tok_per_w, N_EXPERTS), jnp.float32),
            pltpu.VMEM((tok_per_w, N_GROUP), jnp.float32),
            pltpu.VMEM((tok_per_w, TOP_K), jnp.float32),
        ],
        compiler_params=pltpu.CompilerParams(needs_layout_passes=False),
    )(_make_route_body(tok_per_w, n_batch))
    return f(logits, gmax)


def kernel(hidden_states, kernel):
    logits, gmax = _compute_logits(hidden_states, kernel)
    return _route(logits, gmax)
